# Initial kernel scaffold; baseline (speedup 1.0000x reference)
import jax
import jax.numpy as jnp
from jax.experimental import pallas as pl


def kernel(idx_or_len, pe, device):
    raise NotImplementedError("write your pallas kernel here")


# speedup vs baseline: 3.0051x; 3.0051x over previous
import functools

import jax
import jax.numpy as jnp
from jax import lax
from jax.experimental import pallas as pl
from jax.experimental.pallas import tpu as pltpu
from jax.experimental.pallas import tpu_sc as plsc

_B = 4096
_L = 200
_D = 64
_MAX_LEN = 256
_N = _B * _L
_CH = 128
_NC = 2
_NS = 16
_NW = _NC * _NS
_ROWS_PER_W = _N // _NW
_CH_PER_W = _ROWS_PER_W // _CH


@jax.jit
def _pos_gather(idx2d, pe):
    mesh = plsc.VectorSubcoreMesh(core_axis_name="c", subcore_axis_name="s")

    @functools.partial(
        pl.kernel,
        mesh=mesh,
        out_type=jax.ShapeDtypeStruct((_N, _D), jnp.float32),
        compiler_params=pltpu.CompilerParams(use_tc_tiling_on_sc=False),
        scratch_types=[
            pltpu.VMEM((_CH_PER_W, _CH), jnp.int32),
            pltpu.VMEM((2, _CH, _D), jnp.float32),
            pltpu.SemaphoreType.DMA,
            pltpu.SemaphoreType.DMA,
        ],
    )
    def body(idx_hbm, pe_hbm, out_hbm, idx_v, rows_v, gsem, osem):
        wid = lax.axis_index("s") * _NC + lax.axis_index("c")
        cbase = wid * _CH_PER_W
        pltpu.sync_copy(idx_hbm.at[pl.ds(cbase, _CH_PER_W)], idx_v)

        def step(j, carry):
            def clamp16(i, c):
                v = idx_v[j, pl.ds(i * 16, 16)]
                idx_v[j, pl.ds(i * 16, 16)] = jnp.minimum(v, _MAX_LEN - 1)
                return c

            lax.fori_loop(0, _CH // 16, clamp16, 0, unroll=True)

            b = lax.rem(j, 2)
            pltpu.async_copy(pe_hbm.at[idx_v.at[j]], rows_v.at[b], gsem).wait()
            pltpu.sync_copy(rows_v.at[b],
                            out_hbm.at[pl.ds((cbase + j) * _CH, _CH)])
            return carry

        lax.fori_loop(0, _CH_PER_W, step, 0)

    return body(idx2d, pe)


def kernel(idx_or_len, pe, device=0):
    idx2d = idx_or_len.astype(jnp.int32).reshape(_N // _CH, _CH)
    out = _pos_gather(idx2d, pe.astype(jnp.float32))
    return out.reshape(_B, _L, _D)


# speedup vs baseline: 3.0254x; 1.0068x over previous
import functools

import jax
import jax.numpy as jnp
from jax import lax
from jax.experimental import pallas as pl
from jax.experimental.pallas import tpu as pltpu
from jax.experimental.pallas import tpu_sc as plsc

_B = 4096
_L = 200
_D = 64
_MAX_LEN = 256
_N = _B * _L
_CH = 128
_NC = 2
_NS = 16
_NW = _NC * _NS
_ROWS_PER_W = _N // _NW
_CH_PER_W = _ROWS_PER_W // _CH


@jax.jit
def _pos_gather(idx2d, pe):
    mesh = plsc.VectorSubcoreMesh(core_axis_name="c", subcore_axis_name="s")

    nbuf = 6
    lead = 3

    @functools.partial(
        pl.kernel,
        mesh=mesh,
        out_type=jax.ShapeDtypeStruct((_N, _D), jnp.float32),
        compiler_params=pltpu.CompilerParams(use_tc_tiling_on_sc=False),
        scratch_types=[
            pltpu.VMEM((_CH_PER_W, _CH), jnp.int32),
            pltpu.VMEM((nbuf, _CH, _D), jnp.float32),
            pltpu.SemaphoreType.DMA,
            pltpu.SemaphoreType.DMA,
        ],
    )
    def body(idx_hbm, pe_hbm, out_hbm, idx_v, rows_v, gsem, osem):
        wid = lax.axis_index("s") * _NC + lax.axis_index("c")
        cbase = wid * _CH_PER_W
        pltpu.sync_copy(idx_hbm.at[pl.ds(cbase, _CH_PER_W)], idx_v)

        def clamp(j):
            def clamp16(i, c):
                v = idx_v[j, pl.ds(i * 16, 16)]
                idx_v[j, pl.ds(i * 16, 16)] = jnp.minimum(v, _MAX_LEN - 1)
                return c

            lax.fori_loop(0, _CH // 16, clamp16, 0, unroll=True)

        def start_gather(j):
            clamp(j)
            pltpu.async_copy(pe_hbm.at[idx_v.at[j]],
                             rows_v.at[lax.rem(j, nbuf)], gsem)

        def wait_gather(j):
            pltpu.make_async_copy(pe_hbm.at[idx_v.at[j]],
                                  rows_v.at[lax.rem(j, nbuf)], gsem).wait()

        def start_out(j):
            pltpu.async_copy(rows_v.at[lax.rem(j, nbuf)],
                             out_hbm.at[pl.ds((cbase + j) * _CH, _CH)], osem)

        def wait_out(j):
            pltpu.make_async_copy(rows_v.at[lax.rem(j, nbuf)],
                                  out_hbm.at[pl.ds((cbase + j) * _CH, _CH)],
                                  osem).wait()

        for j in range(lead):
            start_gather(j)

        def step(j, carry):
            @pl.when(j >= nbuf - lead)
            def _():
                wait_out(j - (nbuf - lead))

            @pl.when(j + lead < _CH_PER_W)
            def _():
                start_gather(j + lead)

            wait_gather(j)
            start_out(j)
            return carry

        lax.fori_loop(0, _CH_PER_W, step, 0)

        for j in range(_CH_PER_W - (nbuf - lead), _CH_PER_W):
            wait_out(j)

    return body(idx2d, pe)


def kernel(idx_or_len, pe, device=0):
    idx2d = idx_or_len.astype(jnp.int32).reshape(_N // _CH, _CH)
    out = _pos_gather(idx2d, pe.astype(jnp.float32))
    return out.reshape(_B, _L, _D)


# speedup vs baseline: 5.0559x; 1.6711x over previous
import functools

import jax
import jax.numpy as jnp
from jax import lax
from jax.experimental import pallas as pl
from jax.experimental.pallas import tpu as pltpu
from jax.experimental.pallas import tpu_sc as plsc

_B = 4096
_L = 200
_D = 64
_MAX_LEN = 256
_N = _B * _L
_CH = 128
_NC = 2
_NS = 16
_NW = _NC * _NS
_ROWS_PER_W = _N // _NW
_CH_PER_W = _ROWS_PER_W // _CH


@jax.jit
def _pos_gather(idx2d, pe):
    mesh = plsc.VectorSubcoreMesh(core_axis_name="c", subcore_axis_name="s")

    nbuf = 6
    lead = 3

    @functools.partial(
        pl.kernel,
        mesh=mesh,
        out_type=jax.ShapeDtypeStruct((_N, _D), jnp.float32),
        compiler_params=pltpu.CompilerParams(use_tc_tiling_on_sc=False),
        scratch_types=[
            pltpu.VMEM((_CH_PER_W, _CH), jnp.int32),
            pltpu.VMEM((nbuf, _CH, _D), jnp.float32),
            pltpu.VMEM_SHARED((_MAX_LEN, _D), jnp.float32),
            pltpu.SemaphoreType.DMA,
            pltpu.SemaphoreType.DMA,
        ],
    )
    def body(idx_hbm, pe_hbm, out_hbm, idx_v, rows_v, table_sh, gsem, osem):
        wid = lax.axis_index("s") * _NC + lax.axis_index("c")
        cbase = wid * _CH_PER_W

        @pl.when(lax.axis_index("s") == 0)
        def _():
            pltpu.sync_copy(pe_hbm, table_sh)

        pltpu.sync_copy(idx_hbm.at[pl.ds(cbase, _CH_PER_W)], idx_v)
        plsc.subcore_barrier()

        def clamp(j):
            def clamp16(i, c):
                v = idx_v[j, pl.ds(i * 16, 16)]
                idx_v[j, pl.ds(i * 16, 16)] = jnp.minimum(v, _MAX_LEN - 1)
                return c

            lax.fori_loop(0, _CH // 16, clamp16, 0, unroll=True)

        def start_gather(j):
            clamp(j)
            pltpu.async_copy(table_sh.at[idx_v.at[j]],
                             rows_v.at[lax.rem(j, nbuf)], gsem)

        def wait_gather(j):
            pltpu.make_async_copy(table_sh.at[idx_v.at[j]],
                                  rows_v.at[lax.rem(j, nbuf)], gsem).wait()

        def start_out(j):
            pltpu.async_copy(rows_v.at[lax.rem(j, nbuf)],
                             out_hbm.at[pl.ds((cbase + j) * _CH, _CH)], osem)

        def wait_out(j):
            pltpu.make_async_copy(rows_v.at[lax.rem(j, nbuf)],
                                  out_hbm.at[pl.ds((cbase + j) * _CH, _CH)],
                                  osem).wait()

        for j in range(lead):
            start_gather(j)

        def step(j, carry):
            @pl.when(j >= nbuf - lead)
            def _():
                wait_out(j - (nbuf - lead))

            @pl.when(j + lead < _CH_PER_W)
            def _():
                start_gather(j + lead)

            wait_gather(j)
            start_out(j)
            return carry

        lax.fori_loop(0, _CH_PER_W, step, 0)

        for j in range(_CH_PER_W - (nbuf - lead), _CH_PER_W):
            wait_out(j)

    return body(idx2d, pe)


def kernel(idx_or_len, pe, device=0):
    idx2d = idx_or_len.astype(jnp.int32).reshape(_N // _CH, _CH)
    out = _pos_gather(idx2d, pe.astype(jnp.float32))
    return out.reshape(_B, _L, _D)
